# baseline (speedup 1.0000x reference)
import functools

import jax
import jax.numpy as jnp
from jax.experimental import pallas as pl
from jax.experimental.pallas import tpu as pltpu

_L = (
    ("vshift", 32, 128, 64, False),
    ("vshift", 32, 64, 64, True),
    ("vshift", 16, 64, 128, False),
    ("vshift", 16, 128, 128, True),
    ("vshift", 8, 128, 256, False),
    ("vshift", 8, 256, 256, False),
    ("vshift", 8, 256, 256, False),
    ("vshift", 8, 256, 256, True),
    ("vshift", 4, 256, 512, False),
)


def _fill_borders(P, H, W):
    P[:, pl.ds(0, 1), pl.ds(1, W), :] = P[:, pl.ds(2, 1), pl.ds(1, W), :]
    P[:, pl.ds(H + 1, 1), pl.ds(1, W), :] = P[:, pl.ds(H - 1, 1), pl.ds(1, W), :]
    P[:, :, pl.ds(0, 1), :] = P[:, :, pl.ds(2, 1), :]
    P[:, :, pl.ds(W + 1, 1), :] = P[:, :, pl.ds(W - 1, 1), :]


def _conv_stack_kernel(x_ref,
                       w0, b0, w1, b1, w2, b2, w3, b3, w4, b4,
                       w5, b5, w6, b6, w7, b7, w8, b8,
                       o_ref, pad1, pad2, pad3, pad4, pad5, pad6, pad7, pad8,
                       ps_a, ps_b, ps_c, *, BB):
    w_refs = (w0, w1, w2, w3, w4, w5, w6, w7, w8)
    b_refs = (b0, b1, b2, b3, b4, b5, b6, b7, b8)
    pads = (x_ref, pad1, pad2, pad3, pad4, pad5, pad6, pad7, pad8)
    ps = (ps_a, ps_a, ps_a, ps_b, ps_c, ps_c, ps_c, ps_c, ps_c)

    for li, (mode, H, cin, cout, pool) in enumerate(_L):
        P = pads[li]
        PS = ps[li]
        W = H
        if mode == "vshift":
            R = BB * (H + 2) * W
            K = 3 * cin
            for dx in range(3):
                PS[pl.ds(0, R), pl.ds(dx * cin, cin)] = (
                    P[:, :, pl.ds(dx, W), pl.ds(0, cin)].reshape(R, cin))
            y = jnp.dot(PS[pl.ds(0, R), pl.ds(0, K)], w_refs[li][...],
                        preferred_element_type=jnp.float32)
            y = y.reshape(BB, H + 2, W, 3 * cout)
            acc = (y[:, 0:H, :, 0:cout]
                   + y[:, 1:H + 1, :, cout:2 * cout]
                   + y[:, 2:H + 2, :, 2 * cout:3 * cout])
        else:
            R = BB * H * W
            K = 9 * cin
            for tap in range(9):
                dy, dx = divmod(tap, 3)
                PS[pl.ds(0, R), pl.ds(tap * cin, cin)] = (
                    P[:, pl.ds(dy, H), pl.ds(dx, W), :].reshape(R, cin))
            acc = jnp.dot(PS[pl.ds(0, R), pl.ds(0, K)], w_refs[li][...],
                          preferred_element_type=jnp.float32)
            acc = acc.reshape(BB, H, W, cout)

        if pool:
            Ho = H // 2
            a5 = acc.reshape(BB, Ho, 2, W, cout)
            a = jnp.maximum(a5[:, :, 0], a5[:, :, 1])
            b5 = a.reshape(BB, Ho, Ho, 2, cout)
            acc = jnp.maximum(b5[:, :, :, 0], b5[:, :, :, 1])
        else:
            Ho = H
            acc = acc.reshape(BB, H, W, cout)
        r = jnp.maximum(acc + b_refs[li][...], 0.0)
        r = r.astype(jnp.bfloat16)
        if li < 8:
            Pn = pads[li + 1]
            Pn[:, pl.ds(1, Ho), pl.ds(1, Ho), :] = r
            _fill_borders(Pn, Ho, Ho)
        else:
            o_ref[...] = r


def _prep_weights(conv_ws):
    out = []
    for li, (mode, H, cin, cout, pool) in enumerate(_L):
        w = conv_ws[li]
        if li == 0:
            w = jnp.pad(w, ((0, 0), (0, cin - w.shape[1]), (0, 0)))
        if mode == "vshift":
            wc = w.reshape(3, 3, cin, cout).transpose(1, 2, 0, 3)
            out.append(wc.reshape(3 * cin, 3 * cout))
        else:
            out.append(w.reshape(9 * cin, cout))
    return out


def _conv_stack(xp, conv_ws, conv_bs, BB):
    n = xp.shape[0]
    in_specs = [pl.BlockSpec((BB, 34, 40, 128), lambda i: (i, 0, 0, 0))]
    operands = [xp]
    for w, b in zip(conv_ws, conv_bs):
        in_specs.append(pl.BlockSpec(w.shape, lambda i: (0, 0)))
        in_specs.append(pl.BlockSpec(b.shape, lambda i: (0, 0)))
        operands.append(w)
        operands.append(b)
    scratch = [
        pltpu.VMEM((BB, 34, 40, 64), jnp.bfloat16),
        pltpu.VMEM((BB, 18, 24, 64), jnp.bfloat16),
        pltpu.VMEM((BB, 18, 24, 128), jnp.bfloat16),
        pltpu.VMEM((BB, 10, 16, 128), jnp.bfloat16),
        pltpu.VMEM((BB, 10, 16, 256), jnp.bfloat16),
        pltpu.VMEM((BB, 10, 16, 256), jnp.bfloat16),
        pltpu.VMEM((BB, 10, 16, 256), jnp.bfloat16),
        pltpu.VMEM((BB, 6, 12, 256), jnp.bfloat16),
        pltpu.VMEM((BB * 34 * 32, 384), jnp.bfloat16),
        pltpu.VMEM((BB * 18 * 16, 384), jnp.bfloat16),
        pltpu.VMEM((BB * 10 * 8, 768), jnp.bfloat16),
    ]
    return pl.pallas_call(
        functools.partial(_conv_stack_kernel, BB=BB),
        out_shape=jax.ShapeDtypeStruct((n, 4, 4, 512), jnp.bfloat16),
        grid_spec=pltpu.PrefetchScalarGridSpec(
            num_scalar_prefetch=0,
            grid=(n // BB,),
            in_specs=in_specs,
            out_specs=pl.BlockSpec((BB, 4, 4, 512), lambda i: (i, 0, 0, 0)),
            scratch_shapes=scratch,
        ),
        compiler_params=pltpu.CompilerParams(
            dimension_semantics=("parallel",),
            vmem_limit_bytes=64 * 1024 * 1024),
    )(*operands)


def _fc_kernel(a_ref, w_ref, b_ref, o_ref, *, relu):
    r = jnp.dot(a_ref[...], w_ref[...], preferred_element_type=jnp.float32)
    r = r + b_ref[...]
    if relu:
        r = jnp.maximum(r, 0.0)
    o_ref[...] = r.astype(o_ref.dtype)


def _fc(a, w_packed, b, *, relu, out_dtype):
    m, k = a.shape
    n_blocks, kw, tn = w_packed.shape
    n = n_blocks * tn
    return pl.pallas_call(
        functools.partial(_fc_kernel, relu=relu),
        out_shape=jax.ShapeDtypeStruct((m, n), out_dtype),
        grid_spec=pltpu.PrefetchScalarGridSpec(
            num_scalar_prefetch=0,
            grid=(n_blocks,),
            in_specs=[
                pl.BlockSpec((m, k), lambda j: (0, 0)),
                pl.BlockSpec((None, k, tn), lambda j: (j, 0, 0)),
                pl.BlockSpec((1, tn), lambda j: (0, j)),
            ],
            out_specs=pl.BlockSpec((m, tn), lambda j: (0, j)),
        ),
        compiler_params=pltpu.CompilerParams(
            dimension_semantics=("parallel",),
            vmem_limit_bytes=48 * 1024 * 1024),
    )(a, w_packed, b)


def kernel(x, conv0_w, conv0_b, conv1_w, conv1_b, conv2_w, conv2_b,
           conv3_w, conv3_b, conv4_w, conv4_b, conv5_w, conv5_b,
           conv6_w, conv6_b, conv7_w, conv7_b, conv8_w, conv8_b,
           fc1_w, fc1_b, fc2_w, fc2_b, fc3_w, fc3_b):
    x_nhwc = jnp.transpose(x, (0, 2, 3, 1)).astype(jnp.bfloat16)
    xp = jnp.pad(x_nhwc, ((0, 0), (1, 1), (1, 1), (0, 0)), mode="reflect")
    xp = jnp.pad(xp, ((0, 0), (0, 0), (0, 6), (0, 125)))
    conv_ws = _prep_weights((conv0_w, conv1_w, conv2_w, conv3_w, conv4_w,
                             conv5_w, conv6_w, conv7_w, conv8_w))
    conv_bs = (conv0_b, conv1_b, conv2_b, conv3_b, conv4_b,
               conv5_b, conv6_b, conv7_b, conv8_b)
    feat = _conv_stack(xp, conv_ws, conv_bs, BB=8)
    a = feat.reshape(x.shape[0], 8192)
    a = _fc(a, fc1_w, fc1_b, relu=True, out_dtype=jnp.bfloat16)
    a = _fc(a, fc2_w, fc2_b, relu=True, out_dtype=jnp.bfloat16)
    logits = _fc(a, fc3_w, fc3_b, relu=False, out_dtype=jnp.float32)
    return logits[:, :100]


# speedup vs baseline: 1.1214x; 1.1214x over previous
import functools

import jax
import jax.numpy as jnp
from jax.experimental import pallas as pl
from jax.experimental.pallas import tpu as pltpu

_L = (
    (32, 3, 64, False),
    (32, 64, 64, True),
    (16, 64, 128, False),
    (16, 128, 128, True),
    (8, 128, 256, False),
    (8, 256, 256, False),
    (8, 256, 256, False),
    (8, 256, 256, True),
    (4, 256, 512, False),
)


def _fill_borders(P, H, W):
    P[:, pl.ds(0, 1), pl.ds(1, W), :] = P[:, pl.ds(2, 1), pl.ds(1, W), :]
    P[:, pl.ds(H + 1, 1), pl.ds(1, W), :] = P[:, pl.ds(H - 1, 1), pl.ds(1, W), :]
    P[:, :, pl.ds(0, 1), :] = P[:, :, pl.ds(2, 1), :]
    P[:, :, pl.ds(W + 1, 1), :] = P[:, :, pl.ds(W - 1, 1), :]


def _layer(li, P, PS, w_ref, b_ref, P_next, o_ref, BBh, bin0, bout0):
    H, cin, cout, pool = _L[li]
    W = H
    R = BBh * (H + 2) * W
    K = 3 * cin
    for dx in range(3):
        PS[pl.ds(0, R), pl.ds(dx * cin, cin)] = (
            P[pl.ds(bin0, BBh), :, pl.ds(dx, W), pl.ds(0, cin)].reshape(R, cin))
    y = jnp.dot(PS[pl.ds(0, R), pl.ds(0, K)], w_ref[...],
                preferred_element_type=jnp.float32)
    y = y.reshape(BBh, H + 2, W, 3 * cout)
    acc = (y[:, 0:H, :, 0:cout]
           + y[:, 1:H + 1, :, cout:2 * cout]
           + y[:, 2:H + 2, :, 2 * cout:3 * cout])
    if pool:
        Ho = H // 2
        a5 = acc.reshape(BBh, Ho, 2, W, cout)
        a = jnp.maximum(a5[:, :, 0], a5[:, :, 1])
        b5 = a.reshape(BBh, Ho, Ho, 2, cout)
        acc = jnp.maximum(b5[:, :, :, 0], b5[:, :, :, 1])
    else:
        Ho = H
    r = jnp.maximum(acc + b_ref[...], 0.0)
    r = r.astype(jnp.bfloat16)
    if li < 8:
        P_next[:, pl.ds(1, Ho), pl.ds(1, Ho), :] = r
        _fill_borders(P_next, Ho, Ho)
    else:
        o_ref[pl.ds(bout0, BBh)] = r


def _conv_stack_kernel(x_ref,
                       w0, b0, w1, b1, w2, b2, w3, b3, w4, b4,
                       w5, b5, w6, b6, w7, b7, w8, b8,
                       o_ref, *scratch, BB):
    w_refs = (w0, w1, w2, w3, w4, w5, w6, w7, w8)
    b_refs = (b0, b1, b2, b3, b4, b5, b6, b7, b8)
    BBh = BB // 2
    nh = 11
    halves = []
    for h in range(2):
        s = scratch[h * nh:(h + 1) * nh]
        pads = (x_ref,) + tuple(s[0:8])
        ps = (s[8], s[8], s[8], s[9], s[10], s[10], s[10], s[10], s[10])
        halves.append((pads, ps))

    for li in range(9):
        for h in range(2):
            pads, ps = halves[h]
            _layer(li, pads[li], ps[li], w_refs[li], b_refs[li],
                   pads[li + 1] if li < 8 else None,
                   o_ref, BBh,
                   h * BBh if li == 0 else 0,
                   h * BBh)


def _prep_weights(conv_ws):
    out = []
    for li, (H, cin, cout, pool) in enumerate(_L):
        w = conv_ws[li]
        wc = w.reshape(3, 3, cin, cout).transpose(1, 2, 0, 3)
        out.append(wc.reshape(3 * cin, 3 * cout))
    return out


def _conv_stack(xp, conv_ws, conv_bs, BB):
    n = xp.shape[0]
    BBh = BB // 2
    in_specs = [pl.BlockSpec((BB, 34, 40, 128), lambda i: (i, 0, 0, 0))]
    operands = [xp]
    for w, b in zip(conv_ws, conv_bs):
        in_specs.append(pl.BlockSpec(w.shape, lambda i: (0, 0)))
        in_specs.append(pl.BlockSpec(b.shape, lambda i: (0, 0)))
        operands.append(w)
        operands.append(b)
    def half_scratch():
        return [
            pltpu.VMEM((BBh, 34, 40, 64), jnp.bfloat16),
            pltpu.VMEM((BBh, 18, 24, 64), jnp.bfloat16),
            pltpu.VMEM((BBh, 18, 24, 128), jnp.bfloat16),
            pltpu.VMEM((BBh, 10, 16, 128), jnp.bfloat16),
            pltpu.VMEM((BBh, 10, 16, 256), jnp.bfloat16),
            pltpu.VMEM((BBh, 10, 16, 256), jnp.bfloat16),
            pltpu.VMEM((BBh, 10, 16, 256), jnp.bfloat16),
            pltpu.VMEM((BBh, 6, 12, 256), jnp.bfloat16),
            pltpu.VMEM((BBh * 34 * 32, 192), jnp.bfloat16),
            pltpu.VMEM((BBh * 18 * 16, 384), jnp.bfloat16),
            pltpu.VMEM((BBh * 10 * 8, 768), jnp.bfloat16),
        ]
    scratch = half_scratch() + half_scratch()
    return pl.pallas_call(
        functools.partial(_conv_stack_kernel, BB=BB),
        out_shape=jax.ShapeDtypeStruct((n, 4, 4, 512), jnp.bfloat16),
        grid_spec=pltpu.PrefetchScalarGridSpec(
            num_scalar_prefetch=0,
            grid=(n // BB,),
            in_specs=in_specs,
            out_specs=pl.BlockSpec((BB, 4, 4, 512), lambda i: (i, 0, 0, 0)),
            scratch_shapes=scratch,
        ),
        compiler_params=pltpu.CompilerParams(
            dimension_semantics=("parallel",),
            vmem_limit_bytes=64 * 1024 * 1024),
    )(*operands)


def _fc_kernel(a_ref, w_ref, b_ref, o_ref, *, relu):
    r = jnp.dot(a_ref[...], w_ref[...], preferred_element_type=jnp.float32)
    r = r + b_ref[...]
    if relu:
        r = jnp.maximum(r, 0.0)
    o_ref[...] = r.astype(o_ref.dtype)


def _fc(a, w_packed, b, *, relu, out_dtype):
    m, k = a.shape
    n_blocks, kw, tn = w_packed.shape
    n = n_blocks * tn
    return pl.pallas_call(
        functools.partial(_fc_kernel, relu=relu),
        out_shape=jax.ShapeDtypeStruct((m, n), out_dtype),
        grid_spec=pltpu.PrefetchScalarGridSpec(
            num_scalar_prefetch=0,
            grid=(n_blocks,),
            in_specs=[
                pl.BlockSpec((m, k), lambda j: (0, 0)),
                pl.BlockSpec((None, k, tn), lambda j: (j, 0, 0)),
                pl.BlockSpec((1, tn), lambda j: (0, j)),
            ],
            out_specs=pl.BlockSpec((m, tn), lambda j: (0, j)),
        ),
        compiler_params=pltpu.CompilerParams(
            dimension_semantics=("parallel",),
            vmem_limit_bytes=48 * 1024 * 1024),
    )(a, w_packed, b)


def kernel(x, conv0_w, conv0_b, conv1_w, conv1_b, conv2_w, conv2_b,
           conv3_w, conv3_b, conv4_w, conv4_b, conv5_w, conv5_b,
           conv6_w, conv6_b, conv7_w, conv7_b, conv8_w, conv8_b,
           fc1_w, fc1_b, fc2_w, fc2_b, fc3_w, fc3_b):
    x_nhwc = jnp.transpose(x, (0, 2, 3, 1)).astype(jnp.bfloat16)
    xp = jnp.pad(x_nhwc, ((0, 0), (1, 1), (1, 1), (0, 0)), mode="reflect")
    xp = jnp.pad(xp, ((0, 0), (0, 0), (0, 6), (0, 125)))
    conv_ws = _prep_weights((conv0_w, conv1_w, conv2_w, conv3_w, conv4_w,
                             conv5_w, conv6_w, conv7_w, conv8_w))
    conv_bs = (conv0_b, conv1_b, conv2_b, conv3_b, conv4_b,
               conv5_b, conv6_b, conv7_b, conv8_b)
    feat = _conv_stack(xp, conv_ws, conv_bs, BB=8)
    a = feat.reshape(x.shape[0], 8192)
    a = _fc(a, fc1_w, fc1_b, relu=True, out_dtype=jnp.bfloat16)
    a = _fc(a, fc2_w, fc2_b, relu=True, out_dtype=jnp.bfloat16)
    logits = _fc(a, fc3_w, fc3_b, relu=False, out_dtype=jnp.float32)
    return logits[:, :100]


# speedup vs baseline: 1.1881x; 1.0595x over previous
import functools

import jax
import jax.numpy as jnp
from jax.experimental import pallas as pl
from jax.experimental.pallas import tpu as pltpu

_L = (
    (32, 3, 64, False),
    (32, 64, 64, True),
    (16, 64, 128, False),
    (16, 128, 128, True),
    (8, 128, 256, False),
    (8, 256, 256, False),
    (8, 256, 256, False),
    (8, 256, 256, True),
    (4, 256, 512, False),
)


def _fill_borders(P, H, W):
    P[:, pl.ds(0, 1), pl.ds(1, W), :] = P[:, pl.ds(2, 1), pl.ds(1, W), :]
    P[:, pl.ds(H + 1, 1), pl.ds(1, W), :] = P[:, pl.ds(H - 1, 1), pl.ds(1, W), :]
    P[:, :, pl.ds(0, 1), :] = P[:, :, pl.ds(2, 1), :]
    P[:, :, pl.ds(W + 1, 1), :] = P[:, :, pl.ds(W - 1, 1), :]


def _layer(li, P, PS, w_ref, b_ref, P_next, o_ref, BBh, bin0, bout0):
    H, cin, cout, pool = _L[li]
    W = H
    R = BBh * (H + 2) * W
    patch = jnp.concatenate(
        [P[pl.ds(bin0, BBh), :, pl.ds(dx, W), pl.ds(0, cin)].reshape(R, cin)
         for dx in range(3)], axis=1)
    y = jnp.dot(patch, w_ref[...], preferred_element_type=jnp.float32)
    y = y.reshape(BBh, H + 2, W, 3 * cout)
    acc = (y[:, 0:H, :, 0:cout]
           + y[:, 1:H + 1, :, cout:2 * cout]
           + y[:, 2:H + 2, :, 2 * cout:3 * cout])
    if pool:
        Ho = H // 2
        a5 = acc.reshape(BBh, Ho, 2, W, cout)
        a = jnp.maximum(a5[:, :, 0], a5[:, :, 1])
        b5 = a.reshape(BBh, Ho, Ho, 2, cout)
        acc = jnp.maximum(b5[:, :, :, 0], b5[:, :, :, 1])
    else:
        Ho = H
    r = jnp.maximum(acc + b_ref[...], 0.0)
    r = r.astype(jnp.bfloat16)
    if li < 8:
        P_next[:, pl.ds(1, Ho), pl.ds(1, Ho), :] = r
        _fill_borders(P_next, Ho, Ho)
    else:
        o_ref[pl.ds(bout0, BBh)] = r


def _conv_stack_kernel(x_ref,
                       w0, b0, w1, b1, w2, b2, w3, b3, w4, b4,
                       w5, b5, w6, b6, w7, b7, w8, b8,
                       o_ref, *scratch, BB):
    w_refs = (w0, w1, w2, w3, w4, w5, w6, w7, w8)
    b_refs = (b0, b1, b2, b3, b4, b5, b6, b7, b8)
    BBh = BB // 2
    nh = 11
    halves = []
    for h in range(2):
        s = scratch[h * nh:(h + 1) * nh]
        pads = (x_ref,) + tuple(s[0:8])
        ps = (s[8], s[8], s[8], s[9], s[10], s[10], s[10], s[10], s[10])
        halves.append((pads, ps))

    for li in range(9):
        for h in range(2):
            pads, ps = halves[h]
            _layer(li, pads[li], ps[li], w_refs[li], b_refs[li],
                   pads[li + 1] if li < 8 else None,
                   o_ref, BBh,
                   h * BBh if li == 0 else 0,
                   h * BBh)


def _prep_weights(conv_ws):
    out = []
    for li, (H, cin, cout, pool) in enumerate(_L):
        w = conv_ws[li]
        wc = w.reshape(3, 3, cin, cout).transpose(1, 2, 0, 3)
        out.append(wc.reshape(3 * cin, 3 * cout))
    return out


def _conv_stack(xp, conv_ws, conv_bs, BB):
    n = xp.shape[0]
    BBh = BB // 2
    in_specs = [pl.BlockSpec((BB, 34, 40, 128), lambda i: (i, 0, 0, 0))]
    operands = [xp]
    for w, b in zip(conv_ws, conv_bs):
        in_specs.append(pl.BlockSpec(w.shape, lambda i: (0, 0)))
        in_specs.append(pl.BlockSpec(b.shape, lambda i: (0, 0)))
        operands.append(w)
        operands.append(b)
    def half_scratch():
        return [
            pltpu.VMEM((BBh, 34, 40, 64), jnp.bfloat16),
            pltpu.VMEM((BBh, 18, 24, 64), jnp.bfloat16),
            pltpu.VMEM((BBh, 18, 24, 128), jnp.bfloat16),
            pltpu.VMEM((BBh, 10, 16, 128), jnp.bfloat16),
            pltpu.VMEM((BBh, 10, 16, 256), jnp.bfloat16),
            pltpu.VMEM((BBh, 10, 16, 256), jnp.bfloat16),
            pltpu.VMEM((BBh, 10, 16, 256), jnp.bfloat16),
            pltpu.VMEM((BBh, 6, 12, 256), jnp.bfloat16),
            pltpu.VMEM((BBh * 34 * 32, 192), jnp.bfloat16),
            pltpu.VMEM((BBh * 18 * 16, 384), jnp.bfloat16),
            pltpu.VMEM((BBh * 10 * 8, 768), jnp.bfloat16),
        ]
    scratch = half_scratch() + half_scratch()
    return pl.pallas_call(
        functools.partial(_conv_stack_kernel, BB=BB),
        out_shape=jax.ShapeDtypeStruct((n, 4, 4, 512), jnp.bfloat16),
        grid_spec=pltpu.PrefetchScalarGridSpec(
            num_scalar_prefetch=0,
            grid=(n // BB,),
            in_specs=in_specs,
            out_specs=pl.BlockSpec((BB, 4, 4, 512), lambda i: (i, 0, 0, 0)),
            scratch_shapes=scratch,
        ),
        compiler_params=pltpu.CompilerParams(
            dimension_semantics=("parallel",),
            vmem_limit_bytes=64 * 1024 * 1024),
    )(*operands)


def _fc_kernel(a_ref, w_ref, b_ref, o_ref, *, relu):
    r = jnp.dot(a_ref[...], w_ref[...], preferred_element_type=jnp.float32)
    r = r + b_ref[...]
    if relu:
        r = jnp.maximum(r, 0.0)
    o_ref[...] = r.astype(o_ref.dtype)


def _fc(a, w_packed, b, *, relu, out_dtype):
    m, k = a.shape
    n_blocks, kw, tn = w_packed.shape
    n = n_blocks * tn
    return pl.pallas_call(
        functools.partial(_fc_kernel, relu=relu),
        out_shape=jax.ShapeDtypeStruct((m, n), out_dtype),
        grid_spec=pltpu.PrefetchScalarGridSpec(
            num_scalar_prefetch=0,
            grid=(n_blocks,),
            in_specs=[
                pl.BlockSpec((m, k), lambda j: (0, 0)),
                pl.BlockSpec((None, k, tn), lambda j: (j, 0, 0)),
                pl.BlockSpec((1, tn), lambda j: (0, j)),
            ],
            out_specs=pl.BlockSpec((m, tn), lambda j: (0, j)),
        ),
        compiler_params=pltpu.CompilerParams(
            dimension_semantics=("parallel",),
            vmem_limit_bytes=48 * 1024 * 1024),
    )(a, w_packed, b)


def kernel(x, conv0_w, conv0_b, conv1_w, conv1_b, conv2_w, conv2_b,
           conv3_w, conv3_b, conv4_w, conv4_b, conv5_w, conv5_b,
           conv6_w, conv6_b, conv7_w, conv7_b, conv8_w, conv8_b,
           fc1_w, fc1_b, fc2_w, fc2_b, fc3_w, fc3_b):
    x_nhwc = jnp.transpose(x, (0, 2, 3, 1)).astype(jnp.bfloat16)
    xp = jnp.pad(x_nhwc, ((0, 0), (1, 1), (1, 1), (0, 0)), mode="reflect")
    xp = jnp.pad(xp, ((0, 0), (0, 0), (0, 6), (0, 125)))
    conv_ws = _prep_weights((conv0_w, conv1_w, conv2_w, conv3_w, conv4_w,
                             conv5_w, conv6_w, conv7_w, conv8_w))
    conv_bs = (conv0_b, conv1_b, conv2_b, conv3_b, conv4_b,
               conv5_b, conv6_b, conv7_b, conv8_b)
    feat = _conv_stack(xp, conv_ws, conv_bs, BB=8)
    a = feat.reshape(x.shape[0], 8192)
    a = _fc(a, fc1_w, fc1_b, relu=True, out_dtype=jnp.bfloat16)
    a = _fc(a, fc2_w, fc2_b, relu=True, out_dtype=jnp.bfloat16)
    logits = _fc(a, fc3_w, fc3_b, relu=False, out_dtype=jnp.float32)
    return logits[:, :100]


# speedup vs baseline: 1.1891x; 1.0008x over previous
import functools

import jax
import jax.numpy as jnp
from jax.experimental import pallas as pl
from jax.experimental.pallas import tpu as pltpu

_L = (
    (32, 3, 64, False),
    (32, 64, 64, True),
    (16, 64, 128, False),
    (16, 128, 128, True),
    (8, 128, 256, False),
    (8, 256, 256, False),
    (8, 256, 256, False),
    (8, 256, 256, True),
    (4, 256, 512, False),
)


def _fill_borders(P, H, W):
    P[:, pl.ds(0, 1), pl.ds(1, W), :] = P[:, pl.ds(2, 1), pl.ds(1, W), :]
    P[:, pl.ds(H + 1, 1), pl.ds(1, W), :] = P[:, pl.ds(H - 1, 1), pl.ds(1, W), :]
    P[:, :, pl.ds(0, 1), :] = P[:, :, pl.ds(2, 1), :]
    P[:, :, pl.ds(W + 1, 1), :] = P[:, :, pl.ds(W - 1, 1), :]


def _layer(li, P, w_ref, b_ref, P_next, o_ref, BBh, bin0, bout0):
    H, cin, cout, pool = _L[li]
    W = H
    R = BBh * (H + 2) * W
    patch = jnp.concatenate(
        [P[pl.ds(bin0, BBh), :, pl.ds(dx, W), pl.ds(0, cin)].reshape(R, cin)
         for dx in range(3)], axis=1)
    y = jnp.dot(patch, w_ref[...], preferred_element_type=jnp.float32)
    y = y.reshape(BBh, H + 2, W, 3 * cout)
    acc = (y[:, 0:H, :, 0:cout]
           + y[:, 1:H + 1, :, cout:2 * cout]
           + y[:, 2:H + 2, :, 2 * cout:3 * cout])
    if pool:
        Ho = H // 2
        a5 = acc.reshape(BBh, Ho, 2, W, cout)
        a = jnp.maximum(a5[:, :, 0], a5[:, :, 1])
        b5 = a.reshape(BBh, Ho, Ho, 2, cout)
        acc = jnp.maximum(b5[:, :, :, 0], b5[:, :, :, 1])
    else:
        Ho = H
    r = jnp.maximum(acc + b_ref[...], 0.0)
    r = r.astype(jnp.bfloat16)
    if li < 8:
        P_next[:, pl.ds(1, Ho), pl.ds(1, Ho), :] = r
        _fill_borders(P_next, Ho, Ho)
    else:
        o_ref[pl.ds(bout0, BBh)] = r


def _conv_stack_kernel(x_ref,
                       w0, b0, w1, b1, w2, b2, w3, b3, w4, b4,
                       w5, b5, w6, b6, w7, b7, w8, b8,
                       o_ref, *scratch, BB):
    w_refs = (w0, w1, w2, w3, w4, w5, w6, w7, w8)
    b_refs = (b0, b1, b2, b3, b4, b5, b6, b7, b8)
    BBh = BB // 2
    nh = 8
    halves = []
    for h in range(2):
        s = scratch[h * nh:(h + 1) * nh]
        halves.append((x_ref,) + tuple(s[0:8]))

    for li in range(9):
        for h in range(2):
            pads = halves[h]
            _layer(li, pads[li], w_refs[li], b_refs[li],
                   pads[li + 1] if li < 8 else None,
                   o_ref, BBh,
                   h * BBh if li == 0 else 0,
                   h * BBh)


def _prep_weights(conv_ws):
    out = []
    for li, (H, cin, cout, pool) in enumerate(_L):
        w = conv_ws[li]
        wc = w.reshape(3, 3, cin, cout).transpose(1, 2, 0, 3)
        out.append(wc.reshape(3 * cin, 3 * cout))
    return out


def _conv_stack(xp, conv_ws, conv_bs, BB):
    n = xp.shape[0]
    BBh = BB // 2
    in_specs = [pl.BlockSpec((BB, 34, 40, 128), lambda i: (i, 0, 0, 0))]
    operands = [xp]
    for w, b in zip(conv_ws, conv_bs):
        in_specs.append(pl.BlockSpec(w.shape, lambda i: (0, 0)))
        in_specs.append(pl.BlockSpec(b.shape, lambda i: (0, 0)))
        operands.append(w)
        operands.append(b)
    def half_scratch():
        return [
            pltpu.VMEM((BBh, 34, 40, 64), jnp.bfloat16),
            pltpu.VMEM((BBh, 18, 24, 64), jnp.bfloat16),
            pltpu.VMEM((BBh, 18, 24, 128), jnp.bfloat16),
            pltpu.VMEM((BBh, 10, 16, 128), jnp.bfloat16),
            pltpu.VMEM((BBh, 10, 16, 256), jnp.bfloat16),
            pltpu.VMEM((BBh, 10, 16, 256), jnp.bfloat16),
            pltpu.VMEM((BBh, 10, 16, 256), jnp.bfloat16),
            pltpu.VMEM((BBh, 6, 12, 256), jnp.bfloat16),
        ]
    scratch = half_scratch() + half_scratch()
    return pl.pallas_call(
        functools.partial(_conv_stack_kernel, BB=BB),
        out_shape=jax.ShapeDtypeStruct((n, 4, 4, 512), jnp.bfloat16),
        grid_spec=pltpu.PrefetchScalarGridSpec(
            num_scalar_prefetch=0,
            grid=(n // BB,),
            in_specs=in_specs,
            out_specs=pl.BlockSpec((BB, 4, 4, 512), lambda i: (i, 0, 0, 0)),
            scratch_shapes=scratch,
        ),
        compiler_params=pltpu.CompilerParams(
            dimension_semantics=("parallel",),
            vmem_limit_bytes=64 * 1024 * 1024),
    )(*operands)


def _fc_kernel(a_ref, w_ref, b_ref, o_ref, *, relu):
    r = jnp.dot(a_ref[...], w_ref[...], preferred_element_type=jnp.float32)
    r = r + b_ref[...]
    if relu:
        r = jnp.maximum(r, 0.0)
    o_ref[...] = r.astype(o_ref.dtype)


def _fc(a, w_packed, b, *, relu, out_dtype):
    m, k = a.shape
    n_blocks, kw, tn = w_packed.shape
    n = n_blocks * tn
    return pl.pallas_call(
        functools.partial(_fc_kernel, relu=relu),
        out_shape=jax.ShapeDtypeStruct((m, n), out_dtype),
        grid_spec=pltpu.PrefetchScalarGridSpec(
            num_scalar_prefetch=0,
            grid=(n_blocks,),
            in_specs=[
                pl.BlockSpec((m, k), lambda j: (0, 0)),
                pl.BlockSpec((None, k, tn), lambda j: (j, 0, 0)),
                pl.BlockSpec((1, tn), lambda j: (0, j)),
            ],
            out_specs=pl.BlockSpec((m, tn), lambda j: (0, j)),
        ),
        compiler_params=pltpu.CompilerParams(
            dimension_semantics=("parallel",),
            vmem_limit_bytes=48 * 1024 * 1024),
    )(a, w_packed, b)


def kernel(x, conv0_w, conv0_b, conv1_w, conv1_b, conv2_w, conv2_b,
           conv3_w, conv3_b, conv4_w, conv4_b, conv5_w, conv5_b,
           conv6_w, conv6_b, conv7_w, conv7_b, conv8_w, conv8_b,
           fc1_w, fc1_b, fc2_w, fc2_b, fc3_w, fc3_b):
    x_nhwc = jnp.transpose(x, (0, 2, 3, 1)).astype(jnp.bfloat16)
    xp = jnp.pad(x_nhwc, ((0, 0), (1, 1), (1, 1), (0, 0)), mode="reflect")
    xp = jnp.pad(xp, ((0, 0), (0, 0), (0, 6), (0, 125)))
    conv_ws = _prep_weights((conv0_w, conv1_w, conv2_w, conv3_w, conv4_w,
                             conv5_w, conv6_w, conv7_w, conv8_w))
    conv_bs = (conv0_b, conv1_b, conv2_b, conv3_b, conv4_b,
               conv5_b, conv6_b, conv7_b, conv8_b)
    feat = _conv_stack(xp, conv_ws, conv_bs, BB=8)
    a = feat.reshape(x.shape[0], 8192)
    a = _fc(a, fc1_w, fc1_b, relu=True, out_dtype=jnp.bfloat16)
    a = _fc(a, fc2_w, fc2_b, relu=True, out_dtype=jnp.bfloat16)
    logits = _fc(a, fc3_w, fc3_b, relu=False, out_dtype=jnp.float32)
    return logits[:, :100]


# speedup vs baseline: 1.3386x; 1.1257x over previous
import functools

import jax
import jax.numpy as jnp
from jax.experimental import pallas as pl
from jax.experimental.pallas import tpu as pltpu

_L = (
    (32, 3, 64, False),
    (32, 64, 64, True),
    (16, 64, 128, False),
    (16, 128, 128, True),
    (8, 128, 256, False),
    (8, 256, 256, False),
    (8, 256, 256, False),
    (8, 256, 256, True),
    (4, 256, 512, False),
)


def _fill_borders(P, H, W):
    P[:, pl.ds(0, 1), pl.ds(1, W), :] = P[:, pl.ds(2, 1), pl.ds(1, W), :]
    P[:, pl.ds(H + 1, 1), pl.ds(1, W), :] = P[:, pl.ds(H - 1, 1), pl.ds(1, W), :]
    P[:, :, pl.ds(0, 1), :] = P[:, :, pl.ds(2, 1), :]
    P[:, :, pl.ds(W + 1, 1), :] = P[:, :, pl.ds(W - 1, 1), :]


def _layer(li, P, w_ref, b_ref, P_next, o_ref, BBh, bin0, bout0):
    H, cin, cout, pool = _L[li]
    W = H
    R = BBh * (H + 2) * W
    patch = jnp.concatenate(
        [P[pl.ds(bin0, BBh), :, pl.ds(dx, W), pl.ds(0, cin)].reshape(R, cin)
         for dx in range(3)], axis=1)
    y = jnp.dot(patch, w_ref[...], preferred_element_type=jnp.float32)
    y = y.reshape(BBh, H + 2, W, 3 * cout)
    acc = (y[:, 0:H, :, 0:cout]
           + y[:, 1:H + 1, :, cout:2 * cout]
           + y[:, 2:H + 2, :, 2 * cout:3 * cout])
    if pool:
        Ho = H // 2
        a5 = acc.reshape(BBh, Ho, 2, W, cout)
        a = jnp.maximum(a5[:, :, 0], a5[:, :, 1])
        b5 = a.reshape(BBh, Ho, Ho, 2, cout)
        acc = jnp.maximum(b5[:, :, :, 0], b5[:, :, :, 1])
    else:
        Ho = H
    r = jnp.maximum(acc + b_ref[...], 0.0)
    r = r.astype(jnp.bfloat16)
    if li < 8:
        rb = jnp.concatenate([r[:, 1:2], r, r[:, Ho - 2:Ho - 1]], axis=1)
        rb = jnp.concatenate([rb[:, :, 1:2], rb, rb[:, :, Ho - 2:Ho - 1]],
                             axis=2)
        P_next[:, pl.ds(0, Ho + 2), pl.ds(0, Ho + 2), :] = rb
    else:
        o_ref[pl.ds(bout0, BBh)] = r


def _conv_stack_kernel(x_ref,
                       w0, b0, w1, b1, w2, b2, w3, b3, w4, b4,
                       w5, b5, w6, b6, w7, b7, w8, b8,
                       o_ref, *scratch, BB):
    w_refs = (w0, w1, w2, w3, w4, w5, w6, w7, w8)
    b_refs = (b0, b1, b2, b3, b4, b5, b6, b7, b8)
    BBh = BB // 2
    nh = 8
    halves = []
    for h in range(2):
        s = scratch[h * nh:(h + 1) * nh]
        halves.append((x_ref,) + tuple(s[0:8]))

    for li in range(9):
        for h in range(2):
            pads = halves[h]
            _layer(li, pads[li], w_refs[li], b_refs[li],
                   pads[li + 1] if li < 8 else None,
                   o_ref, BBh,
                   h * BBh if li == 0 else 0,
                   h * BBh)


def _prep_weights(conv_ws):
    out = []
    for li, (H, cin, cout, pool) in enumerate(_L):
        w = conv_ws[li]
        wc = w.reshape(3, 3, cin, cout).transpose(1, 2, 0, 3)
        out.append(wc.reshape(3 * cin, 3 * cout))
    return out


def _conv_stack(xp, conv_ws, conv_bs, BB):
    n = xp.shape[0]
    BBh = BB // 2
    in_specs = [pl.BlockSpec((BB, 34, 40, 128), lambda i: (i, 0, 0, 0))]
    operands = [xp]
    for w, b in zip(conv_ws, conv_bs):
        in_specs.append(pl.BlockSpec(w.shape, lambda i: (0, 0)))
        in_specs.append(pl.BlockSpec(b.shape, lambda i: (0, 0)))
        operands.append(w)
        operands.append(b)
    def half_scratch():
        return [
            pltpu.VMEM((BBh, 34, 40, 64), jnp.bfloat16),
            pltpu.VMEM((BBh, 18, 24, 64), jnp.bfloat16),
            pltpu.VMEM((BBh, 18, 24, 128), jnp.bfloat16),
            pltpu.VMEM((BBh, 10, 16, 128), jnp.bfloat16),
            pltpu.VMEM((BBh, 10, 16, 256), jnp.bfloat16),
            pltpu.VMEM((BBh, 10, 16, 256), jnp.bfloat16),
            pltpu.VMEM((BBh, 10, 16, 256), jnp.bfloat16),
            pltpu.VMEM((BBh, 6, 12, 256), jnp.bfloat16),
        ]
    scratch = half_scratch() + half_scratch()
    return pl.pallas_call(
        functools.partial(_conv_stack_kernel, BB=BB),
        out_shape=jax.ShapeDtypeStruct((n, 4, 4, 512), jnp.bfloat16),
        grid_spec=pltpu.PrefetchScalarGridSpec(
            num_scalar_prefetch=0,
            grid=(n // BB,),
            in_specs=in_specs,
            out_specs=pl.BlockSpec((BB, 4, 4, 512), lambda i: (i, 0, 0, 0)),
            scratch_shapes=scratch,
        ),
        compiler_params=pltpu.CompilerParams(
            dimension_semantics=("parallel",),
            vmem_limit_bytes=64 * 1024 * 1024),
    )(*operands)


def _fc_kernel(a_ref, w_ref, b_ref, o_ref, *, relu):
    r = jnp.dot(a_ref[...], w_ref[...], preferred_element_type=jnp.float32)
    r = r + b_ref[...]
    if relu:
        r = jnp.maximum(r, 0.0)
    o_ref[...] = r.astype(o_ref.dtype)


def _fc(a, w_packed, b, *, relu, out_dtype):
    m, k = a.shape
    n_blocks, kw, tn = w_packed.shape
    n = n_blocks * tn
    return pl.pallas_call(
        functools.partial(_fc_kernel, relu=relu),
        out_shape=jax.ShapeDtypeStruct((m, n), out_dtype),
        grid_spec=pltpu.PrefetchScalarGridSpec(
            num_scalar_prefetch=0,
            grid=(n_blocks,),
            in_specs=[
                pl.BlockSpec((m, k), lambda j: (0, 0)),
                pl.BlockSpec((None, k, tn), lambda j: (j, 0, 0)),
                pl.BlockSpec((1, tn), lambda j: (0, j)),
            ],
            out_specs=pl.BlockSpec((m, tn), lambda j: (0, j)),
        ),
        compiler_params=pltpu.CompilerParams(
            dimension_semantics=("parallel",),
            vmem_limit_bytes=48 * 1024 * 1024),
    )(a, w_packed, b)


def kernel(x, conv0_w, conv0_b, conv1_w, conv1_b, conv2_w, conv2_b,
           conv3_w, conv3_b, conv4_w, conv4_b, conv5_w, conv5_b,
           conv6_w, conv6_b, conv7_w, conv7_b, conv8_w, conv8_b,
           fc1_w, fc1_b, fc2_w, fc2_b, fc3_w, fc3_b):
    x_nhwc = jnp.transpose(x, (0, 2, 3, 1)).astype(jnp.bfloat16)
    xp = jnp.pad(x_nhwc, ((0, 0), (1, 1), (1, 1), (0, 0)), mode="reflect")
    xp = jnp.pad(xp, ((0, 0), (0, 0), (0, 6), (0, 125)))
    conv_ws = _prep_weights((conv0_w, conv1_w, conv2_w, conv3_w, conv4_w,
                             conv5_w, conv6_w, conv7_w, conv8_w))
    conv_bs = (conv0_b, conv1_b, conv2_b, conv3_b, conv4_b,
               conv5_b, conv6_b, conv7_b, conv8_b)
    feat = _conv_stack(xp, conv_ws, conv_bs, BB=8)
    a = feat.reshape(x.shape[0], 8192)
    a = _fc(a, fc1_w, fc1_b, relu=True, out_dtype=jnp.bfloat16)
    a = _fc(a, fc2_w, fc2_b, relu=True, out_dtype=jnp.bfloat16)
    logits = _fc(a, fc3_w, fc3_b, relu=False, out_dtype=jnp.float32)
    return logits[:, :100]


# speedup vs baseline: 1.4856x; 1.1098x over previous
import functools

import jax
import jax.numpy as jnp
from jax.experimental import pallas as pl
from jax.experimental.pallas import tpu as pltpu

_L = (
    (32, 3, 64, False),
    (32, 64, 64, True),
    (16, 64, 128, False),
    (16, 128, 128, True),
    (8, 128, 256, False),
    (8, 256, 256, False),
    (8, 256, 256, False),
    (8, 256, 256, True),
    (4, 256, 512, False),
)


def _fill_borders(P, H, W):
    P[:, pl.ds(0, 1), pl.ds(1, W), :] = P[:, pl.ds(2, 1), pl.ds(1, W), :]
    P[:, pl.ds(H + 1, 1), pl.ds(1, W), :] = P[:, pl.ds(H - 1, 1), pl.ds(1, W), :]
    P[:, :, pl.ds(0, 1), :] = P[:, :, pl.ds(2, 1), :]
    P[:, :, pl.ds(W + 1, 1), :] = P[:, :, pl.ds(W - 1, 1), :]


def _layer(li, P, w_ref, b_ref, BBh):
    H, cin, cout, pool = _L[li]
    W = H
    R = BBh * (H + 2) * W
    patch = jnp.concatenate(
        [P[:, :, dx:dx + W, 0:cin].reshape(R, cin) for dx in range(3)],
        axis=1)
    y = jnp.dot(patch, w_ref[...], preferred_element_type=jnp.float32)
    y = y.reshape(BBh, H + 2, W, 3 * cout)
    acc = (y[:, 0:H, :, 0:cout]
           + y[:, 1:H + 1, :, cout:2 * cout]
           + y[:, 2:H + 2, :, 2 * cout:3 * cout])
    if pool:
        Ho = H // 2
        a5 = acc.reshape(BBh, Ho, 2, W, cout)
        a = jnp.maximum(a5[:, :, 0], a5[:, :, 1])
        b5 = a.reshape(BBh, Ho, Ho, 2, cout)
        acc = jnp.maximum(b5[:, :, :, 0], b5[:, :, :, 1])
    else:
        Ho = H
    r = jnp.maximum(acc + b_ref[...], 0.0)
    r = r.astype(jnp.bfloat16)
    if li < 8:
        rb = jnp.concatenate([r[:, 1:2], r, r[:, Ho - 2:Ho - 1]], axis=1)
        rb = jnp.concatenate([rb[:, :, 1:2], rb, rb[:, :, Ho - 2:Ho - 1]],
                             axis=2)
        return rb
    return r


def _conv_stack_kernel(x_ref,
                       w0, b0, w1, b1, w2, b2, w3, b3, w4, b4,
                       w5, b5, w6, b6, w7, b7, w8, b8,
                       o_ref, *scratch, BB):
    w_refs = (w0, w1, w2, w3, w4, w5, w6, w7, w8)
    b_refs = (b0, b1, b2, b3, b4, b5, b6, b7, b8)
    BBh = BB // 2
    vals = [x_ref[pl.ds(h * BBh, BBh)] for h in range(2)]
    for li in range(9):
        for h in range(2):
            vals[h] = _layer(li, vals[h], w_refs[li], b_refs[li], BBh)
    for h in range(2):
        o_ref[pl.ds(h * BBh, BBh)] = vals[h]


def _prep_weights(conv_ws):
    out = []
    for li, (H, cin, cout, pool) in enumerate(_L):
        w = conv_ws[li]
        wc = w.reshape(3, 3, cin, cout).transpose(1, 2, 0, 3)
        out.append(wc.reshape(3 * cin, 3 * cout))
    return out


def _conv_stack(xp, conv_ws, conv_bs, BB):
    n = xp.shape[0]
    BBh = BB // 2
    in_specs = [pl.BlockSpec((BB, 34, 40, 128), lambda i: (i, 0, 0, 0))]
    operands = [xp]
    for w, b in zip(conv_ws, conv_bs):
        in_specs.append(pl.BlockSpec(w.shape, lambda i: (0, 0)))
        in_specs.append(pl.BlockSpec(b.shape, lambda i: (0, 0)))
        operands.append(w)
        operands.append(b)
    return pl.pallas_call(
        functools.partial(_conv_stack_kernel, BB=BB),
        out_shape=jax.ShapeDtypeStruct((n, 4, 4, 512), jnp.bfloat16),
        grid_spec=pltpu.PrefetchScalarGridSpec(
            num_scalar_prefetch=0,
            grid=(n // BB,),
            in_specs=in_specs,
            out_specs=pl.BlockSpec((BB, 4, 4, 512), lambda i: (i, 0, 0, 0)),
        ),
        compiler_params=pltpu.CompilerParams(
            dimension_semantics=("parallel",),
            vmem_limit_bytes=64 * 1024 * 1024),
    )(*operands)


def _fc_kernel(a_ref, w_ref, b_ref, o_ref, *, relu):
    r = jnp.dot(a_ref[...], w_ref[...], preferred_element_type=jnp.float32)
    r = r + b_ref[...]
    if relu:
        r = jnp.maximum(r, 0.0)
    o_ref[...] = r.astype(o_ref.dtype)


def _fc(a, w_packed, b, *, relu, out_dtype):
    m, k = a.shape
    n_blocks, kw, tn = w_packed.shape
    n = n_blocks * tn
    return pl.pallas_call(
        functools.partial(_fc_kernel, relu=relu),
        out_shape=jax.ShapeDtypeStruct((m, n), out_dtype),
        grid_spec=pltpu.PrefetchScalarGridSpec(
            num_scalar_prefetch=0,
            grid=(n_blocks,),
            in_specs=[
                pl.BlockSpec((m, k), lambda j: (0, 0)),
                pl.BlockSpec((None, k, tn), lambda j: (j, 0, 0)),
                pl.BlockSpec((1, tn), lambda j: (0, j)),
            ],
            out_specs=pl.BlockSpec((m, tn), lambda j: (0, j)),
        ),
        compiler_params=pltpu.CompilerParams(
            dimension_semantics=("parallel",),
            vmem_limit_bytes=48 * 1024 * 1024),
    )(a, w_packed, b)


def kernel(x, conv0_w, conv0_b, conv1_w, conv1_b, conv2_w, conv2_b,
           conv3_w, conv3_b, conv4_w, conv4_b, conv5_w, conv5_b,
           conv6_w, conv6_b, conv7_w, conv7_b, conv8_w, conv8_b,
           fc1_w, fc1_b, fc2_w, fc2_b, fc3_w, fc3_b):
    x_nhwc = jnp.transpose(x, (0, 2, 3, 1)).astype(jnp.bfloat16)
    xp = jnp.pad(x_nhwc, ((0, 0), (1, 1), (1, 1), (0, 0)), mode="reflect")
    xp = jnp.pad(xp, ((0, 0), (0, 0), (0, 6), (0, 125)))
    conv_ws = _prep_weights((conv0_w, conv1_w, conv2_w, conv3_w, conv4_w,
                             conv5_w, conv6_w, conv7_w, conv8_w))
    conv_bs = (conv0_b, conv1_b, conv2_b, conv3_b, conv4_b,
               conv5_b, conv6_b, conv7_b, conv8_b)
    feat = _conv_stack(xp, conv_ws, conv_bs, BB=8)
    a = feat.reshape(x.shape[0], 8192)
    a = _fc(a, fc1_w, fc1_b, relu=True, out_dtype=jnp.bfloat16)
    a = _fc(a, fc2_w, fc2_b, relu=True, out_dtype=jnp.bfloat16)
    logits = _fc(a, fc3_w, fc3_b, relu=False, out_dtype=jnp.float32)
    return logits[:, :100]
